# baseline (speedup 1.0000x reference)
import functools

import jax
import jax.numpy as jnp
from jax import lax
from jax.experimental import pallas as pl
from jax.experimental.pallas import tpu as pltpu
from jax.experimental.pallas import tpu_sc as plsc

_B, _F, _V = 16384, 26, 100000
_DC, _DF = 32, 16
_DO = _DC + _DF

_NC, _NS, _L = 2, 16, 16
_NW = _NC * _NS
_NB = 1024
_NCHUNK = _B // _NB
_ITEMS = _F * _NCHUNK
_IPW = _ITEMS // _NW
_G = 128
_NG = _NB // _G


def _sc_body(x_hbm, tab_hbm, feat_hbm, out_hbm, idx_v, rows_v, blk_v,
             feat_v, sem):
    wid = lax.axis_index("s") * _NC + lax.axis_index("c")

    pltpu.sync_copy(feat_hbm, feat_v)
    iota = lax.iota(jnp.int32, _L)

    def _item(k, carry):
        t = wid * _IPW + k
        f = t // _NCHUNK
        b0 = (t % _NCHUNK) * _NB
        pltpu.sync_copy(x_hbm.at[pl.ds(f * _B + b0, _NB)], idx_v)
        fbase = f * _V
        for l in range(_NB // _L):
            sl = pl.ds(l * _L, _L)
            idx_v[sl] = idx_v[sl] + fbase
        cps = []
        for j in range(_NG):
            cps.append(pltpu.async_copy(
                tab_hbm.at[idx_v.at[pl.ds(j * _G, _G)]],
                rows_v.at[pl.ds(j * _G, _G)],
                sem))
        for cp in cps:
            cp.wait()
        def _tr_grp(q, c):
            row = q * _L + iota
            for d in range(_DC):
                v = plsc.load_gather(rows_v, [row, jnp.full((_L,), d, jnp.int32)])
                blk_v[d, pl.ds(q * _L, _L)] = v
            return c

        lax.fori_loop(0, _NB // _L, _tr_grp, 0)
        fcol = jnp.full((_L,), f, jnp.int32)
        for df in range(_DF):
            v = plsc.load_gather(feat_v, [jnp.full((_L,), df, jnp.int32), fcol])
            for l in range(_NB // _L):
                blk_v[_DC + df, pl.ds(l * _L, _L)] = v
        pltpu.sync_copy(blk_v, out_hbm.at[f, :, pl.ds(b0, _NB)])
        return carry

    lax.fori_loop(0, _IPW, _item, 0)


@jax.jit
def kernel(x, tables, feat_table):
    x_rm = x.astype(jnp.int32).T.reshape(_F * _B)
    tab2 = tables.reshape(_F * _V, _DC)
    feat_t = feat_table.T

    mesh = plsc.VectorSubcoreMesh(core_axis_name="c", subcore_axis_name="s")
    out = pl.kernel(
        _sc_body,
        mesh=mesh,
        out_type=jax.ShapeDtypeStruct((_F, _DO, _B), jnp.float32),
        compiler_params=pltpu.CompilerParams(use_tc_tiling_on_sc=False,
                                             needs_layout_passes=False),
        scratch_types=[
            pltpu.VMEM((_NB,), jnp.int32),
            pltpu.VMEM((_NB, _DC), jnp.float32),
            pltpu.VMEM((_DO, _NB), jnp.float32),
            pltpu.VMEM((_DF, 100), jnp.float32),
            pltpu.SemaphoreType.DMA,
        ],
    )(x_rm, tab2, feat_t)
    return out.transpose(2, 0, 1)


# speedup vs baseline: 1.8185x; 1.8185x over previous
import functools

import jax
import jax.numpy as jnp
from jax import lax
from jax.experimental import pallas as pl
from jax.experimental.pallas import tpu as pltpu
from jax.experimental.pallas import tpu_sc as plsc

_B, _F, _V = 16384, 26, 100000
_DC, _DF = 32, 16
_DO = _DC + _DF

_NC, _NS, _L = 2, 16, 16
_NW = _NC * _NS
_NB = 1024
_NCHUNK = _B // _NB
_ITEMS = _F * _NCHUNK
_IPW = _ITEMS // _NW
_G = 128
_NG = _NB // _G

def _tc_transpose_body(in_ref, out_ref):
    blk = in_ref[0]
    parts = [blk[:, j * (_V // 4):(j + 1) * (_V // 4)] for j in range(4)]
    out_ref[...] = jnp.concatenate(parts, axis=0).T


def _tc_transpose(tab_t):
    return pl.pallas_call(
        _tc_transpose_body,
        grid=(_F,),
        in_specs=[pl.BlockSpec((1, _DC, _V), lambda f: (f, 0, 0))],
        out_specs=pl.BlockSpec((_V // 4, _DC * 4), lambda f: (f, 0)),
        out_shape=jax.ShapeDtypeStruct((_F * _V // 4, _DC * 4), jnp.float32),
    )(tab_t)


_SUB = 256
_NSUB = _NB // _SUB
_Q = _V // 4


def _sc_body(x_hbm, tab_hbm, feat_hbm, out_hbm, idx_v, col_v, rows_v,
             blk_v, feat_v, sem):
    wid = lax.axis_index("s") * _NC + lax.axis_index("c")

    pltpu.sync_copy(feat_hbm, feat_v)
    iota = lax.iota(jnp.int32, _L)

    def _item(k, carry):
        t = wid * _IPW + k
        f = t // _NCHUNK
        b0 = (t % _NCHUNK) * _NB
        pltpu.sync_copy(x_hbm.at[pl.ds(f * _B + b0, _NB)], idx_v)
        fbase = f * _Q
        for l in range(_NB // _L):
            sl = pl.ds(l * _L, _L)
            xv = idx_v[sl]
            col_v[sl] = (xv // _Q) * _DC
            idx_v[sl] = lax.rem(xv, _Q) + fbase
        for s in range(_NSUB):
            cps = []
            for j in range(2):
                cps.append(pltpu.async_copy(
                    tab_hbm.at[idx_v.at[pl.ds(s * _SUB + j * _G, _G)]],
                    rows_v.at[pl.ds(j * _G, _G)],
                    sem))
            for cp in cps:
                cp.wait()

            def _tr_grp(q, c):
                row = q * _L + iota
                cbase = col_v[pl.ds(s * _SUB + q * _L, _L)]
                for d in range(_DC):
                    v = plsc.load_gather(rows_v, [row, cbase + d])
                    blk_v[d, pl.ds(s * _SUB + q * _L, _L)] = v
                return c

            lax.fori_loop(0, _SUB // _L, _tr_grp, 0)
        fcol = jnp.full((_L,), f, jnp.int32)
        for df in range(_DF):
            v = plsc.load_gather(feat_v, [jnp.full((_L,), df, jnp.int32), fcol])
            for l in range(_NB // _L):
                blk_v[_DC + df, pl.ds(l * _L, _L)] = v
        pltpu.sync_copy(blk_v, out_hbm.at[f, :, pl.ds(b0, _NB)])
        return carry

    lax.fori_loop(0, _IPW, _item, 0)


@jax.jit
def kernel(x, tables, feat_table):
    x_rm = x.astype(jnp.int32).T.reshape(_F * _B)
    tab_t = tables.transpose(0, 2, 1)
    tab2 = _tc_transpose(tab_t)
    feat_t = jnp.pad(feat_table.T, ((0, 0), (0, 28)))

    mesh = plsc.VectorSubcoreMesh(core_axis_name="c", subcore_axis_name="s")
    out = pl.kernel(
        _sc_body,
        mesh=mesh,
        out_type=jax.ShapeDtypeStruct((_F, _DO, _B), jnp.float32),
        compiler_params=pltpu.CompilerParams(use_tc_tiling_on_sc=False,
                                             needs_layout_passes=False),
        scratch_types=[
            pltpu.VMEM((_NB,), jnp.int32),
            pltpu.VMEM((_NB,), jnp.int32),
            pltpu.VMEM((_SUB, _DC * 4), jnp.float32),
            pltpu.VMEM((_DO, _NB), jnp.float32),
            pltpu.VMEM((_DF, 128), jnp.float32),
            pltpu.SemaphoreType.DMA,
        ],
    )(x_rm, tab2, feat_t)
    return out.transpose(2, 0, 1)


# speedup vs baseline: 1.9865x; 1.0924x over previous
import functools

import jax
import jax.numpy as jnp
from jax import lax
from jax.experimental import pallas as pl
from jax.experimental.pallas import tpu as pltpu
from jax.experimental.pallas import tpu_sc as plsc

_B, _F, _V = 16384, 26, 100000
_DC, _DF = 32, 16
_DO = _DC + _DF

_NC, _NS, _L = 2, 16, 16
_NW = _NC * _NS
_NB = 512
_NCHUNK = _B // _NB
_ITEMS = _F * _NCHUNK
_IPW = _ITEMS // _NW

def _tc_transpose_body(in_ref, out_ref):
    blk = in_ref[0]
    parts = [blk[:, j * (_V // 4):(j + 1) * (_V // 4)] for j in range(4)]
    out_ref[...] = jnp.concatenate(parts, axis=0).T


def _tc_transpose(tab_t):
    return pl.pallas_call(
        _tc_transpose_body,
        grid=(_F,),
        in_specs=[pl.BlockSpec((1, _DC, _V), lambda f: (f, 0, 0))],
        out_specs=pl.BlockSpec((_V // 4, _DC * 4), lambda f: (f, 0)),
        out_shape=jax.ShapeDtypeStruct((_F * _V // 4, _DC * 4), jnp.float32),
    )(tab_t)


_SUB = 256
_Q = _V // 4


def _sc_body(x_hbm, tab_hbm, feat_hbm, out_hbm, idx_v, col_v, rows_v,
             blk_v, feat_v, sem, osem):
    wid = lax.axis_index("s") * _NC + lax.axis_index("c")

    pltpu.sync_copy(feat_hbm, feat_v)
    iota = lax.iota(jnp.int32, _L)

    def _item(k, carry):
        t = wid * _IPW + k
        f = t // _NCHUNK
        b0 = (t % _NCHUNK) * _NB
        pltpu.sync_copy(x_hbm.at[pl.ds(f * _B + b0, _NB)], idx_v)
        fbase = f * _Q
        for l in range(_NB // _L):
            sl = pl.ds(l * _L, _L)
            xv = idx_v[sl]
            col_v[sl] = (xv // _Q) * _DC
            idx_v[sl] = lax.rem(xv, _Q) + fbase
        cps = [pltpu.async_copy(
            tab_hbm.at[idx_v.at[pl.ds(s * _SUB, _SUB)]],
            rows_v.at[pl.ds(s * _SUB, _SUB)],
            sem) for s in range(2)]
        @pl.when(k > 0)
        def _():
            pltpu.make_async_copy(
                blk_v, out_hbm.at[0, :, pl.ds(0, _NB)], osem).wait()

        for s in range(2):
            cps[s].wait()

            def _tr_grp(q, c):
                qq = s * (_SUB // _L) + q
                row = qq * _L + iota
                cbase = col_v[pl.ds(qq * _L, _L)]
                for d in range(_DC):
                    v = plsc.load_gather(rows_v, [row, cbase + d])
                    blk_v[d, pl.ds(qq * _L, _L)] = v
                return c

            lax.fori_loop(0, _SUB // _L, _tr_grp, 0)
        fcol = jnp.full((_L,), f, jnp.int32)
        for df in range(_DF):
            v = plsc.load_gather(feat_v, [jnp.full((_L,), df, jnp.int32), fcol])
            for l in range(_NB // _L):
                blk_v[_DC + df, pl.ds(l * _L, _L)] = v
        pltpu.async_copy(blk_v, out_hbm.at[f, :, pl.ds(b0, _NB)], osem)
        return carry

    lax.fori_loop(0, _IPW, _item, 0)
    pltpu.make_async_copy(blk_v, out_hbm.at[0, :, pl.ds(0, _NB)], osem).wait()


@jax.jit
def kernel(x, tables, feat_table):
    x_rm = x.astype(jnp.int32).T.reshape(_F * _B)
    tab_t = tables.transpose(0, 2, 1)
    tab2 = _tc_transpose(tab_t)
    feat_t = jnp.pad(feat_table.T, ((0, 0), (0, 28)))

    mesh = plsc.VectorSubcoreMesh(core_axis_name="c", subcore_axis_name="s")
    out = pl.kernel(
        _sc_body,
        mesh=mesh,
        out_type=jax.ShapeDtypeStruct((_F, _DO, _B), jnp.float32),
        compiler_params=pltpu.CompilerParams(use_tc_tiling_on_sc=False,
                                             needs_layout_passes=False),
        scratch_types=[
            pltpu.VMEM((_NB,), jnp.int32),
            pltpu.VMEM((_NB,), jnp.int32),
            pltpu.VMEM((_NB, _DC * 4), jnp.float32),
            pltpu.VMEM((_DO, _NB), jnp.float32),
            pltpu.VMEM((_DF, 128), jnp.float32),
            pltpu.SemaphoreType.DMA,
            pltpu.SemaphoreType.DMA,
        ],
    )(x_rm, tab2, feat_t)
    return out.transpose(2, 0, 1)


# speedup vs baseline: 2.2246x; 1.1199x over previous
import functools

import jax
import jax.numpy as jnp
from jax import lax
from jax.experimental import pallas as pl
from jax.experimental.pallas import tpu as pltpu
from jax.experimental.pallas import tpu_sc as plsc

_B, _F, _V = 16384, 26, 100000
_DC, _DF = 32, 16
_DO = _DC + _DF

_NC, _NS, _L = 2, 16, 16
_NW = _NC * _NS
_NB = 256
_NCHUNK = _B // _NB
_ITEMS = _F * _NCHUNK
_IPW = _ITEMS // _NW

def _tc_transpose_body(in_ref, out_ref):
    blk = in_ref[0]
    parts = [blk[:, j * (_V // 4):(j + 1) * (_V // 4)] for j in range(4)]
    out_ref[...] = jnp.concatenate(parts, axis=0).T


def _tc_transpose(tab_t):
    return pl.pallas_call(
        _tc_transpose_body,
        grid=(_F,),
        in_specs=[pl.BlockSpec((1, _DC, _V), lambda f: (f, 0, 0))],
        out_specs=pl.BlockSpec((_V // 4, _DC * 4), lambda f: (f, 0)),
        out_shape=jax.ShapeDtypeStruct((_F * _V // 4, _DC * 4), jnp.float32),
    )(tab_t)


_Q = _V // 4


def _sc_body(x_hbm, tab_hbm, feat_hbm, out_hbm, idx_v, col_v, rows_v,
             blk_v, feat_v, isem, gsem0, gsem1, osem):
    wid = lax.axis_index("s") * _NC + lax.axis_index("c")
    gsems = [gsem0, gsem1]

    pltpu.sync_copy(feat_hbm, feat_v)
    iota = lax.iota(jnp.int32, _L)

    def _src(t):
        f = t // _NCHUNK
        return f, (t % _NCHUNK) * _NB

    def _fire_idx(t, p):
        f, b0 = _src(t)
        pltpu.async_copy(x_hbm.at[pl.ds(f * _B + b0, _NB)],
                         idx_v.at[p], isem)

    def _fire_gather(t, p):
        pltpu.make_async_copy(x_hbm.at[pl.ds(0, _NB)], idx_v.at[p],
                              isem).wait()
        f, _ = _src(t)
        fbase = f * _Q
        for l in range(_NB // _L):
            sl = pl.ds(l * _L, _L)
            xv = idx_v[p, sl]
            col_v[p, sl] = (xv // _Q) * _DC
            idx_v[p, sl] = lax.rem(xv, _Q) + fbase
        pltpu.async_copy(tab_hbm.at[idx_v.at[p]], rows_v.at[p], gsems[p])

    def _consume(t, p):
        pltpu.make_async_copy(tab_hbm.at[pl.ds(0, _NB)], rows_v.at[p],
                              gsems[p]).wait()
        f, b0 = _src(t)
        pv = jnp.full((_L,), p, jnp.int32)

        def _tr_grp(q, c):
            row = q * _L + iota
            cbase = col_v[p, pl.ds(q * _L, _L)]
            for d in range(_DC):
                v = plsc.load_gather(rows_v, [pv, row, cbase + d])
                blk_v[d, pl.ds(q * _L, _L)] = v
            return c

        lax.fori_loop(0, _NB // _L, _tr_grp, 0)
        fcol = jnp.full((_L,), f, jnp.int32)
        for df in range(_DF):
            v = plsc.load_gather(feat_v, [jnp.full((_L,), df, jnp.int32), fcol])
            for l in range(_NB // _L):
                blk_v[_DC + df, pl.ds(l * _L, _L)] = v
        pltpu.async_copy(blk_v, out_hbm.at[f, :, pl.ds(b0, _NB)], osem)

    def _wait_out():
        pltpu.make_async_copy(
            blk_v, out_hbm.at[0, :, pl.ds(0, _NB)], osem).wait()

    base = wid * _IPW
    _fire_idx(base, 0)
    _fire_gather(base, 0)
    _fire_idx(base + 1, 1)

    def _pair(k2, carry):
        k = 2 * k2 + 1
        _fire_gather(base + k, 1)

        @pl.when(k + 1 < _IPW)
        def _():
            _fire_idx(base + k + 1, 0)

        @pl.when(k > 1)
        def _():
            _wait_out()
        _consume(base + k - 1, 0)

        @pl.when(k + 1 < _IPW)
        def _():
            _fire_gather(base + k + 1, 0)

            @pl.when(k + 2 < _IPW)
            def _():
                _fire_idx(base + k + 2, 1)
            _wait_out()
            _consume(base + k, 1)
        return carry

    lax.fori_loop(0, (_IPW + 1) // 2, _pair, 0)
    _wait_out()
    _consume(base + _IPW - 1, (_IPW - 1) % 2)
    _wait_out()


@jax.jit
def kernel(x, tables, feat_table):
    x_rm = x.astype(jnp.int32).T.reshape(_F * _B)
    tab_t = tables.transpose(0, 2, 1)
    tab2 = _tc_transpose(tab_t)
    feat_t = jnp.pad(feat_table.T, ((0, 0), (0, 28)))

    mesh = plsc.VectorSubcoreMesh(core_axis_name="c", subcore_axis_name="s")
    out = pl.kernel(
        _sc_body,
        mesh=mesh,
        out_type=jax.ShapeDtypeStruct((_F, _DO, _B), jnp.float32),
        compiler_params=pltpu.CompilerParams(use_tc_tiling_on_sc=False,
                                             needs_layout_passes=False),
        scratch_types=[
            pltpu.VMEM((2, _NB), jnp.int32),
            pltpu.VMEM((2, _NB), jnp.int32),
            pltpu.VMEM((2, _NB, _DC * 4), jnp.float32),
            pltpu.VMEM((_DO, _NB), jnp.float32),
            pltpu.VMEM((_DF, 128), jnp.float32),
            pltpu.SemaphoreType.DMA,
            pltpu.SemaphoreType.DMA,
            pltpu.SemaphoreType.DMA,
            pltpu.SemaphoreType.DMA,
        ],
    )(x_rm, tab2, feat_t)
    return out.transpose(2, 0, 1)


# speedup vs baseline: 2.5830x; 1.1611x over previous
import functools

import jax
import jax.numpy as jnp
from jax import lax
from jax.experimental import pallas as pl
from jax.experimental.pallas import tpu as pltpu
from jax.experimental.pallas import tpu_sc as plsc

_B, _F, _V = 16384, 26, 100000
_DC, _DF = 32, 16
_DO = _DC + _DF

_NC, _NS, _L = 2, 16, 16
_NW = _NC * _NS
_NB = 256
_NCHUNK = _B // _NB
_ITEMS = _F * _NCHUNK
_IPW = _ITEMS // _NW

def _tc_transpose_body(in_ref, out_ref):
    blk = in_ref[0]
    parts = [blk[:, j * (_V // 4):(j + 1) * (_V // 4)] for j in range(4)]
    out_ref[...] = jnp.concatenate(parts, axis=0).T


def _tc_transpose(tab_t):
    return pl.pallas_call(
        _tc_transpose_body,
        grid=(_F,),
        in_specs=[pl.BlockSpec((1, _DC, _V), lambda f: (f, 0, 0))],
        out_specs=pl.BlockSpec((_V // 4, _DC * 4), lambda f: (f, 0)),
        out_shape=jax.ShapeDtypeStruct((_F * _V // 4, _DC * 4), jnp.float32),
    )(tab_t)


_Q = _V // 4


def _sc_body(x_hbm, tab_hbm, feat_hbm, out_hbm, idx_v, col_v, rows_v,
             blk_v, feat_v, isem, gsem0, gsem1, osem):
    wid = lax.axis_index("s") * _NC + lax.axis_index("c")
    gsems = [gsem0, gsem1]

    pltpu.sync_copy(feat_hbm, feat_v)
    iota = lax.iota(jnp.int32, _L)

    def _src(t):
        f = t // _NCHUNK
        return f, (t % _NCHUNK) * _NB

    def _fire_idx(t, p):
        f, b0 = _src(t)
        pltpu.async_copy(x_hbm.at[pl.ds(f * _B + b0, _NB)],
                         idx_v.at[p], isem)

    def _fire_gather(t, p):
        pltpu.make_async_copy(x_hbm.at[pl.ds(0, _NB)], idx_v.at[p],
                              isem).wait()
        f, _ = _src(t)
        fbase = f * _Q
        for l in range(_NB // _L):
            sl = pl.ds(l * _L, _L)
            xv = idx_v[p, sl]
            col_v[p, sl] = (xv // _Q) * _DC
            idx_v[p, sl] = lax.rem(xv, _Q) + fbase
        pltpu.async_copy(tab_hbm.at[idx_v.at[p]], rows_v.at[p], gsems[p])

    def _consume(t, p):
        pltpu.make_async_copy(tab_hbm.at[pl.ds(0, _NB)], rows_v.at[p],
                              gsems[p]).wait()
        f, b0 = _src(t)
        pv = jnp.full((_L,), p, jnp.int32)
        dlocs = [jnp.bitwise_and(iota + c, _L - 1) for c in range(_L)]

        def _tr_grp(q, c):
            row = q * _L + iota
            cbase = col_v[p, pl.ds(q * _L, _L)]
            for h in range(2):
                for cs in range(_L):
                    dv = h * _L + dlocs[cs]
                    vals = plsc.load_gather(rows_v, [pv, row, cbase + dv])
                    plsc.store_scatter(blk_v, [dv, row], vals)
            return c

        lax.fori_loop(0, _NB // _L, _tr_grp, 0)
        fcol = jnp.full((_L,), f, jnp.int32)
        for df in range(_DF):
            v = plsc.load_gather(feat_v, [jnp.full((_L,), df, jnp.int32), fcol])
            for l in range(_NB // _L):
                blk_v[_DC + df, pl.ds(l * _L, _L)] = v
        pltpu.async_copy(blk_v, out_hbm.at[f, :, pl.ds(b0, _NB)], osem)

    def _wait_out():
        pltpu.make_async_copy(
            blk_v, out_hbm.at[0, :, pl.ds(0, _NB)], osem).wait()

    base = wid * _IPW
    _fire_idx(base, 0)
    _fire_gather(base, 0)
    _fire_idx(base + 1, 1)

    def _pair(k2, carry):
        k = 2 * k2 + 1
        _fire_gather(base + k, 1)

        @pl.when(k + 1 < _IPW)
        def _():
            _fire_idx(base + k + 1, 0)

        @pl.when(k > 1)
        def _():
            _wait_out()
        _consume(base + k - 1, 0)

        @pl.when(k + 1 < _IPW)
        def _():
            _fire_gather(base + k + 1, 0)

            @pl.when(k + 2 < _IPW)
            def _():
                _fire_idx(base + k + 2, 1)
            _wait_out()
            _consume(base + k, 1)
        return carry

    lax.fori_loop(0, (_IPW + 1) // 2, _pair, 0)
    _wait_out()
    _consume(base + _IPW - 1, (_IPW - 1) % 2)
    _wait_out()


@jax.jit
def kernel(x, tables, feat_table):
    x_rm = x.astype(jnp.int32).T.reshape(_F * _B)
    tab_t = tables.transpose(0, 2, 1)
    tab2 = _tc_transpose(tab_t)
    feat_t = jnp.pad(feat_table.T, ((0, 0), (0, 28)))

    mesh = plsc.VectorSubcoreMesh(core_axis_name="c", subcore_axis_name="s")
    out = pl.kernel(
        _sc_body,
        mesh=mesh,
        out_type=jax.ShapeDtypeStruct((_F, _DO, _B), jnp.float32),
        compiler_params=pltpu.CompilerParams(use_tc_tiling_on_sc=False,
                                             needs_layout_passes=False),
        scratch_types=[
            pltpu.VMEM((2, _NB), jnp.int32),
            pltpu.VMEM((2, _NB), jnp.int32),
            pltpu.VMEM((2, _NB, _DC * 4), jnp.float32),
            pltpu.VMEM((_DO, _NB), jnp.float32),
            pltpu.VMEM((_DF, 128), jnp.float32),
            pltpu.SemaphoreType.DMA,
            pltpu.SemaphoreType.DMA,
            pltpu.SemaphoreType.DMA,
            pltpu.SemaphoreType.DMA,
        ],
    )(x_rm, tab2, feat_t)
    return out.transpose(2, 0, 1)


# speedup vs baseline: 2.9986x; 1.1609x over previous
import functools

import jax
import jax.numpy as jnp
from jax import lax
from jax.experimental import pallas as pl
from jax.experimental.pallas import tpu as pltpu
from jax.experimental.pallas import tpu_sc as plsc

_B, _F, _V = 16384, 26, 100000
_DC, _DF = 32, 16
_DO = _DC + _DF

_NC, _NS, _L = 2, 16, 16
_NW = _NC * _NS
_NB = 256
_NCHUNK = _B // _NB
_ITEMS = _F * _NCHUNK
_IPW = _ITEMS // _NW

def _tc_transpose_body(in_ref, out_ref):
    blk = in_ref[0]
    parts = [blk[:, j * (_V // 4):(j + 1) * (_V // 4)] for j in range(4)]
    out_ref[...] = jnp.concatenate(parts, axis=0).T


def _tc_transpose(tab_t):
    return pl.pallas_call(
        _tc_transpose_body,
        grid=(_F,),
        in_specs=[pl.BlockSpec((1, _DC, _V), lambda f: (f, 0, 0))],
        out_specs=pl.BlockSpec((_V // 4, _DC * 4), lambda f: (f, 0)),
        out_shape=jax.ShapeDtypeStruct((_F * _V // 4, _DC * 4), jnp.float32),
    )(tab_t)


_Q = _V // 4


def _sc_body(x_hbm, tab_hbm, feat_hbm, out_hbm, idx_v, col_v, rows_v,
             blk_v, feat_v, isem, gsem0, gsem1, osem):
    wid = lax.axis_index("s") * _NC + lax.axis_index("c")
    gsems = [gsem0, gsem1]

    pltpu.sync_copy(feat_hbm, feat_v)
    iota = lax.iota(jnp.int32, _L)

    def _src(t):
        f = t // _NCHUNK
        return f, (t % _NCHUNK) * _NB

    def _fire_idx(t, p):
        f, b0 = _src(t)
        pltpu.async_copy(x_hbm.at[pl.ds(f * _B + b0, _NB)],
                         idx_v.at[p], isem)

    def _fire_gather(t, p):
        pltpu.make_async_copy(x_hbm.at[pl.ds(0, _NB)], idx_v.at[p],
                              isem).wait()
        f, _ = _src(t)
        fbase = f * _Q
        for l in range(_NB // _L):
            sl = pl.ds(l * _L, _L)
            xv = idx_v[p, sl]
            col_v[p, sl] = (xv // _Q) * _DC
            idx_v[p, sl] = lax.rem(xv, _Q) + fbase
        pltpu.async_copy(tab_hbm.at[idx_v.at[p]], rows_v.at[p], gsems[p])

    def _consume(t, p):
        pltpu.make_async_copy(tab_hbm.at[pl.ds(0, _NB)], rows_v.at[p],
                              gsems[p]).wait()
        f, b0 = _src(t)
        pv = jnp.full((_L,), p, jnp.int32)
        dlocs = [jnp.bitwise_and(iota + c, _L - 1) for c in range(_L)]

        def _tr_grp(q, c):
            row = q * _L + iota
            bt = lax.shift_right_logical(row, 7)
            bl = jnp.bitwise_and(row, 127)
            cbase = col_v[p, pl.ds(q * _L, _L)]
            for h in range(2):
                for cs in range(_L):
                    dv = h * _L + dlocs[cs]
                    vals = plsc.load_gather(rows_v, [pv, row, cbase + dv])
                    plsc.store_scatter(
                        blk_v,
                        [lax.shift_right_logical(dv, 3), bt,
                         jnp.bitwise_and(dv, 7), bl],
                        vals)
            return c

        lax.fori_loop(0, _NB // _L, _tr_grp, 0)
        fcol = jnp.full((_L,), f, jnp.int32)
        for df in range(_DF):
            v = plsc.load_gather(feat_v, [jnp.full((_L,), df, jnp.int32), fcol])
            dt, dr = 4 + df // 8, df % 8
            for bt in range(_NB // 128):
                for l in range(8):
                    blk_v[dt, bt, dr, pl.ds(l * _L, _L)] = v
        pltpu.async_copy(blk_v, out_hbm.at[f, :, pl.ds(b0 // 128, _NB // 128)],
                         osem)

    def _wait_out():
        pltpu.make_async_copy(
            blk_v, out_hbm.at[0, :, pl.ds(0, _NB // 128)], osem).wait()

    base = wid * _IPW
    _fire_idx(base, 0)
    _fire_gather(base, 0)
    _fire_idx(base + 1, 1)

    def _pair(k2, carry):
        k = 2 * k2 + 1
        _fire_gather(base + k, 1)

        @pl.when(k + 1 < _IPW)
        def _():
            _fire_idx(base + k + 1, 0)

        @pl.when(k > 1)
        def _():
            _wait_out()
        _consume(base + k - 1, 0)

        @pl.when(k + 1 < _IPW)
        def _():
            _fire_gather(base + k + 1, 0)

            @pl.when(k + 2 < _IPW)
            def _():
                _fire_idx(base + k + 2, 1)
            _wait_out()
            _consume(base + k, 1)
        return carry

    lax.fori_loop(0, (_IPW + 1) // 2, _pair, 0)
    _wait_out()
    _consume(base + _IPW - 1, (_IPW - 1) % 2)
    _wait_out()


@jax.jit
def kernel(x, tables, feat_table):
    x_rm = x.astype(jnp.int32).T.reshape(_F * _B)
    tab_t = tables.transpose(0, 2, 1)
    tab2 = _tc_transpose(tab_t)
    feat_t = jnp.pad(feat_table.T, ((0, 0), (0, 28)))

    mesh = plsc.VectorSubcoreMesh(core_axis_name="c", subcore_axis_name="s")
    out = pl.kernel(
        _sc_body,
        mesh=mesh,
        out_type=jax.ShapeDtypeStruct((_F, _DO // 8, _B // 128, 8, 128),
                                      jnp.float32),
        compiler_params=pltpu.CompilerParams(use_tc_tiling_on_sc=False,
                                             needs_layout_passes=False),
        scratch_types=[
            pltpu.VMEM((2, _NB), jnp.int32),
            pltpu.VMEM((2, _NB), jnp.int32),
            pltpu.VMEM((2, _NB, _DC * 4), jnp.float32),
            pltpu.VMEM((_DO // 8, _NB // 128, 8, 128), jnp.float32),
            pltpu.VMEM((_DF, 128), jnp.float32),
            pltpu.SemaphoreType.DMA,
            pltpu.SemaphoreType.DMA,
            pltpu.SemaphoreType.DMA,
            pltpu.SemaphoreType.DMA,
        ],
    )(x_rm, tab2, feat_t)
    return out.transpose(2, 4, 0, 1, 3).reshape(_B, _F, _DO)


# speedup vs baseline: 3.1663x; 1.0559x over previous
import functools

import jax
import jax.numpy as jnp
from jax import lax
from jax.experimental import pallas as pl
from jax.experimental.pallas import tpu as pltpu
from jax.experimental.pallas import tpu_sc as plsc

_B, _F, _V = 16384, 26, 100000
_DC, _DF = 32, 16
_DO = _DC + _DF

_NC, _NS, _L = 2, 16, 16
_NW = _NC * _NS
_NB = 256
_NCHUNK = _B // _NB
_ITEMS = _F * _NCHUNK
_IPW = _ITEMS // _NW

def _tc_transpose_body(in_ref, out_ref):
    blk = in_ref[0]
    parts = [blk[:, j * (_V // 4):(j + 1) * (_V // 4)] for j in range(4)]
    out_ref[...] = jnp.concatenate(parts, axis=0).T


def _tc_transpose(tab_t):
    return pl.pallas_call(
        _tc_transpose_body,
        grid=(_F,),
        in_specs=[pl.BlockSpec((1, _DC, _V), lambda f: (f, 0, 0))],
        out_specs=pl.BlockSpec((_V // 4, _DC * 4), lambda f: (f, 0)),
        out_shape=jax.ShapeDtypeStruct((_F * _V // 4, _DC * 4), jnp.float32),
    )(tab_t)


_Q = _V // 4


def _sc_body(x_hbm, tab_hbm, feat_hbm, out_hbm, idx_v, rows_v,
             blk_v, feat_v, isem, gsem0, gsem1, osem):
    wid = lax.axis_index("s") * _NC + lax.axis_index("c")
    gsems = [gsem0, gsem1]

    pltpu.sync_copy(feat_hbm, feat_v)
    iota = lax.iota(jnp.int32, _L)

    def _src(t):
        f = t // _NCHUNK
        return f, (t % _NCHUNK) * _NB

    def _fire_idx(t, p):
        f, b0 = _src(t)
        pltpu.async_copy(x_hbm.at[pl.ds(f * _B + b0, _NB)],
                         idx_v.at[p], isem)

    def _fire_gather(t, p):
        pltpu.make_async_copy(x_hbm.at[pl.ds(0, _NB)], idx_v.at[p],
                              isem).wait()
        f, _ = _src(t)
        fbase = f * _Q
        for l in range(_NB // _L):
            sl = pl.ds(l * _L, _L)
            xv = idx_v[p, sl]
            idx_v[p, sl] = (lax.rem(xv, _Q) + fbase) * 4 + xv // _Q
        pltpu.async_copy(tab_hbm.at[idx_v.at[p]], rows_v.at[p], gsems[p])

    def _consume(t, p):
        pltpu.make_async_copy(tab_hbm.at[pl.ds(0, _NB)], rows_v.at[p],
                              gsems[p]).wait()
        f, b0 = _src(t)
        pv = jnp.full((_L,), p, jnp.int32)
        dlocs = [jnp.bitwise_and(iota + c, _L - 1) for c in range(_L)]

        def _tr_grp(q, c):
            row = q * _L + iota
            bt = lax.shift_right_logical(row, 7)
            bl = jnp.bitwise_and(row, 127)
            for h in range(2):
                for cs in range(_L):
                    dv = h * _L + dlocs[cs]
                    vals = plsc.load_gather(rows_v, [pv, row, dv])
                    plsc.store_scatter(
                        blk_v,
                        [lax.shift_right_logical(dv, 3), bt,
                         jnp.bitwise_and(dv, 7), bl],
                        vals)
            return c

        lax.fori_loop(0, _NB // _L, _tr_grp, 0)
        fcol = jnp.full((_L,), f, jnp.int32)
        for df in range(_DF):
            v = plsc.load_gather(feat_v, [jnp.full((_L,), df, jnp.int32), fcol])
            dt, dr = 4 + df // 8, df % 8
            for bt in range(_NB // 128):
                for l in range(8):
                    blk_v[dt, bt, dr, pl.ds(l * _L, _L)] = v
        pltpu.async_copy(blk_v, out_hbm.at[f, :, pl.ds(b0 // 128, _NB // 128)],
                         osem)

    def _wait_out():
        pltpu.make_async_copy(
            blk_v, out_hbm.at[0, :, pl.ds(0, _NB // 128)], osem).wait()

    base = wid * _IPW
    _fire_idx(base, 0)
    _fire_gather(base, 0)
    _fire_idx(base + 1, 1)

    def _pair(k2, carry):
        k = 2 * k2 + 1
        _fire_gather(base + k, 1)

        @pl.when(k + 1 < _IPW)
        def _():
            _fire_idx(base + k + 1, 0)

        @pl.when(k > 1)
        def _():
            _wait_out()
        _consume(base + k - 1, 0)

        @pl.when(k + 1 < _IPW)
        def _():
            _fire_gather(base + k + 1, 0)

            @pl.when(k + 2 < _IPW)
            def _():
                _fire_idx(base + k + 2, 1)
            _wait_out()
            _consume(base + k, 1)
        return carry

    lax.fori_loop(0, (_IPW + 1) // 2, _pair, 0)
    _wait_out()
    _consume(base + _IPW - 1, (_IPW - 1) % 2)
    _wait_out()


@jax.jit
def kernel(x, tables, feat_table):
    x_rm = x.astype(jnp.int32).T.reshape(_F * _B)
    tab_t = tables.transpose(0, 2, 1)
    tab2 = _tc_transpose(tab_t).reshape(_F * _V, _DC)
    feat_t = jnp.pad(feat_table.T, ((0, 0), (0, 28)))

    mesh = plsc.VectorSubcoreMesh(core_axis_name="c", subcore_axis_name="s")
    out = pl.kernel(
        _sc_body,
        mesh=mesh,
        out_type=jax.ShapeDtypeStruct((_F, _DO // 8, _B // 128, 8, 128),
                                      jnp.float32),
        compiler_params=pltpu.CompilerParams(use_tc_tiling_on_sc=False,
                                             needs_layout_passes=False),
        scratch_types=[
            pltpu.VMEM((2, _NB), jnp.int32),
            pltpu.VMEM((2, _NB, _DC), jnp.float32),
            pltpu.VMEM((_DO // 8, _NB // 128, 8, 128), jnp.float32),
            pltpu.VMEM((_DF, 128), jnp.float32),
            pltpu.SemaphoreType.DMA,
            pltpu.SemaphoreType.DMA,
            pltpu.SemaphoreType.DMA,
            pltpu.SemaphoreType.DMA,
        ],
    )(x_rm, tab2, feat_t)
    return out.transpose(2, 4, 0, 1, 3).reshape(_B, _F, _DO)


# speedup vs baseline: 3.3884x; 1.0702x over previous
import functools

import jax
import jax.numpy as jnp
from jax import lax
from jax.experimental import pallas as pl
from jax.experimental.pallas import tpu as pltpu
from jax.experimental.pallas import tpu_sc as plsc

_B, _F, _V = 16384, 26, 100000
_DC, _DF = 32, 16
_DO = _DC + _DF

_NC, _NS, _L = 2, 16, 16
_NW = _NC * _NS
_NB = 256
_NCHUNK = _B // _NB
_ITEMS = _F * _NCHUNK
_IPW = _ITEMS // _NW

def _tc_transpose_body(in_ref, out_ref):
    blk = in_ref[0]
    parts = [blk[:, j * (_V // 4):(j + 1) * (_V // 4)] for j in range(4)]
    out_ref[...] = jnp.concatenate(parts, axis=0).T


def _tc_transpose(tab_t, f0, nf):
    return pl.pallas_call(
        _tc_transpose_body,
        grid=(nf,),
        in_specs=[pl.BlockSpec((1, _DC, _V), lambda f, f0=f0: (f0 + f, 0, 0))],
        out_specs=pl.BlockSpec((_V // 4, _DC * 4), lambda f: (f, 0)),
        out_shape=jax.ShapeDtypeStruct((nf * _V // 4, _DC * 4), jnp.float32),
    )(tab_t)


_Q = _V // 4


def _sc_body(f0, nf, x_hbm, tab_hbm, feat_hbm, out_hbm, idx_v, rows_v,
             blk_v, feat_v, isem, gsem0, gsem1, osem):
    ipw = nf * _NCHUNK // _NW
    wid = lax.axis_index("s") * _NC + lax.axis_index("c")
    gsems = [gsem0, gsem1]

    pltpu.sync_copy(feat_hbm, feat_v)
    iota = lax.iota(jnp.int32, _L)

    def _src(t):
        f = t // _NCHUNK
        return f, (t % _NCHUNK) * _NB

    def _fire_idx(t, p):
        f, b0 = _src(t)
        pltpu.async_copy(x_hbm.at[pl.ds(f * _B + b0, _NB)],
                         idx_v.at[p], isem)

    def _fire_gather(t, p):
        pltpu.make_async_copy(x_hbm.at[pl.ds(0, _NB)], idx_v.at[p],
                              isem).wait()
        f, _ = _src(t)
        fbase = f * _Q
        for l in range(_NB // _L):
            sl = pl.ds(l * _L, _L)
            xv = idx_v[p, sl]
            idx_v[p, sl] = (lax.rem(xv, _Q) + fbase) * 4 + xv // _Q
        pltpu.async_copy(tab_hbm.at[idx_v.at[p]], rows_v.at[p], gsems[p])

    def _consume(t, p):
        pltpu.make_async_copy(tab_hbm.at[pl.ds(0, _NB)], rows_v.at[p],
                              gsems[p]).wait()
        f, b0 = _src(t)
        pv = jnp.full((_L,), p, jnp.int32)
        dlocs = [jnp.bitwise_and(iota + c, _L - 1) for c in range(_L)]

        def _tr_grp(q, c):
            row = q * _L + iota
            bt = lax.shift_right_logical(row, 7)
            bl = jnp.bitwise_and(row, 127)
            for h in range(2):
                for cs in range(_L):
                    dv = h * _L + dlocs[cs]
                    vals = plsc.load_gather(rows_v, [pv, row, dv])
                    plsc.store_scatter(
                        blk_v,
                        [lax.shift_right_logical(dv, 3), bt,
                         jnp.bitwise_and(dv, 7), bl],
                        vals)
            return c

        lax.fori_loop(0, _NB // _L, _tr_grp, 0)
        fcol = jnp.full((_L,), f0 + f, jnp.int32)
        for df in range(_DF):
            v = plsc.load_gather(feat_v, [jnp.full((_L,), df, jnp.int32), fcol])
            dt, dr = 4 + df // 8, df % 8
            for bt in range(_NB // 128):
                for l in range(8):
                    blk_v[dt, bt, dr, pl.ds(l * _L, _L)] = v
        pltpu.async_copy(blk_v, out_hbm.at[f, :, pl.ds(b0 // 128, _NB // 128)],
                         osem)

    def _wait_out():
        pltpu.make_async_copy(
            blk_v, out_hbm.at[0, :, pl.ds(0, _NB // 128)], osem).wait()

    base = wid * ipw
    _fire_idx(base, 0)
    _fire_gather(base, 0)
    _fire_idx(base + 1, 1)

    def _pair(k2, carry):
        k = 2 * k2 + 1
        _fire_gather(base + k, 1)

        @pl.when(k + 1 < ipw)
        def _():
            _fire_idx(base + k + 1, 0)

        @pl.when(k > 1)
        def _():
            _wait_out()
        _consume(base + k - 1, 0)

        @pl.when(k + 1 < ipw)
        def _():
            _fire_gather(base + k + 1, 0)

            @pl.when(k + 2 < ipw)
            def _():
                _fire_idx(base + k + 2, 1)
            _wait_out()
            _consume(base + k, 1)
        return carry

    lax.fori_loop(0, (ipw + 1) // 2, _pair, 0)
    _wait_out()
    _consume(base + ipw - 1, (ipw - 1) % 2)
    _wait_out()


_HALF = _F // 2


@jax.jit
def kernel(x, tables, feat_table):
    x_rm = x.astype(jnp.int32).T.reshape(_F * _B)
    tab_t = tables.transpose(0, 2, 1)
    feat_t = jnp.pad(feat_table.T, ((0, 0), (0, 28)))

    mesh = plsc.VectorSubcoreMesh(core_axis_name="c", subcore_axis_name="s")
    outs = []
    for f0 in (0, _HALF):
        tab2 = _tc_transpose(tab_t, f0, _HALF).reshape(_HALF * _V, _DC)
        outs.append(pl.kernel(
            functools.partial(_sc_body, f0, _HALF),
            mesh=mesh,
            out_type=jax.ShapeDtypeStruct((_HALF, _DO // 8, _B // 128, 8, 128),
                                          jnp.float32),
            compiler_params=pltpu.CompilerParams(use_tc_tiling_on_sc=False,
                                                 needs_layout_passes=False),
            scratch_types=[
                pltpu.VMEM((2, _NB), jnp.int32),
                pltpu.VMEM((2, _NB, _DC), jnp.float32),
                pltpu.VMEM((_DO // 8, _NB // 128, 8, 128), jnp.float32),
                pltpu.VMEM((_DF, 128), jnp.float32),
                pltpu.SemaphoreType.DMA,
                pltpu.SemaphoreType.DMA,
                pltpu.SemaphoreType.DMA,
                pltpu.SemaphoreType.DMA,
            ],
        )(x_rm[f0 * _B:(f0 + _HALF) * _B], tab2, feat_t))
    out = jnp.concatenate(outs, axis=0)
    return out.transpose(2, 4, 0, 1, 3).reshape(_B, _F, _DO)
